# Initial kernel scaffold; baseline (speedup 1.0000x reference)
import jax
import jax.numpy as jnp
from jax.experimental import pallas as pl


def kernel(x_drug, x_protein, edge_index, edge_type, offset_drug, offset_protein, comp, basis, root, bias):
    raise NotImplementedError("write your pallas kernel here")


# speedup vs baseline: 14.6306x; 14.6306x over previous
import functools

import jax
import jax.numpy as jnp
from jax import lax
from jax.experimental import pallas as pl
from jax.experimental.pallas import tpu as pltpu
from jax.experimental.pallas import tpu_sc as plsc

N_NODES = 10000
E_EDGES = 320000
D_FEAT = 128
R_REL = 8
NB_BASES = 8
NKEY = N_NODES * R_REL

NUM_CORES = 2
NUM_SUBCORES = 16
NUM_WORKERS = NUM_CORES * NUM_SUBCORES
EPT = E_EDGES // NUM_WORKERS
CHUNK = 80
NCHUNK = EPT // CHUNK
LANES = 16

BLKN = 2000
NBLK = N_NODES // BLKN

_MESH = dict(core_axis_name="c", subcore_axis_name="s",
             num_cores=NUM_CORES, num_subcores=NUM_SUBCORES)
_SC_PARAMS = pltpu.CompilerParams(needs_layout_passes=False)


def _cnt_body(kidx_hbm, zeros_hbm, cnt_out, kidx2d, ones_v, cnt_bounce,
              cnt_sh):
    c = lax.axis_index("c")
    s = lax.axis_index("s")
    wid = c * NUM_SUBCORES + s
    for g in range(CHUNK // LANES):
        ones_v[pl.ds(g * LANES, LANES)] = jnp.full((LANES,), 1.0, jnp.float32)

    @pl.when(s == 0)
    def _():
        pltpu.sync_copy(zeros_hbm, cnt_sh)

    plsc.subcore_barrier()
    base = wid * EPT

    def chunk(j, carry):
        off = base + j * CHUNK
        pltpu.sync_copy(kidx_hbm.at[pl.ds(off, CHUNK)], kidx2d.at[0])
        pltpu.sync_copy(ones_v, cnt_sh.at[kidx2d.at[0]], add=True)
        return carry

    lax.fori_loop(0, NCHUNK, chunk, 0)
    plsc.subcore_barrier()
    per_tile = NKEY // NUM_SUBCORES
    pltpu.sync_copy(cnt_sh.at[pl.ds(s * per_tile, per_tile)], cnt_bounce)
    pltpu.sync_copy(cnt_bounce,
                    cnt_out.at[pl.ds(c * NKEY + s * per_tile, per_tile)])


def _count_call(kidx, zeros_nk):
    k = functools.partial(
        pl.kernel,
        out_type=jax.ShapeDtypeStruct((NUM_CORES * NKEY,), jnp.float32),
        mesh=plsc.VectorSubcoreMesh(**_MESH),
        scratch_types=[
            pltpu.VMEM((1, CHUNK), jnp.int32),
            pltpu.VMEM((CHUNK,), jnp.float32),
            pltpu.VMEM((NKEY // NUM_SUBCORES,), jnp.float32),
            pltpu.VMEM_SHARED((NKEY,), jnp.float32),
        ],
        compiler_params=_SC_PARAMS,
    )(_cnt_body)
    return k(kidx, zeros_nk)


def _msg_body(xw_hbm, gidx_hbm, kidx_hbm, norm_hbm, zeros_hbm, agg_out,
              gidx_v, kidx_v, nrm_v, dst2d, msg_v, out_bounce,
              normtab_sh, agg_sh):
    c = lax.axis_index("c")
    s = lax.axis_index("s")
    wid = c * NUM_SUBCORES + s

    @pl.when(s == 0)
    def _():
        pltpu.sync_copy(norm_hbm, normtab_sh)
        pltpu.sync_copy(zeros_hbm, agg_sh)

    plsc.subcore_barrier()
    base = wid * EPT

    def chunk(j, carry):
        off = base + j * CHUNK
        pltpu.sync_copy(gidx_hbm.at[pl.ds(off, CHUNK)], gidx_v)
        pltpu.sync_copy(kidx_hbm.at[pl.ds(off, CHUNK)], kidx_v)
        pltpu.sync_copy(xw_hbm.at[gidx_v], msg_v)
        pltpu.sync_copy(normtab_sh.at[kidx_v], nrm_v)
        for g in range(CHUNK // LANES):
            kv = kidx_v[pl.ds(g * LANES, LANES)]
            nv = nrm_v[pl.ds(g * LANES, LANES)]
            dst2d[0, pl.ds(g * LANES, LANES)] = kv >> 3
            for i in range(LANES):
                snorm = nv[i]
                row = g * LANES + i
                for t in range(D_FEAT // LANES):
                    sl = pl.ds(t * LANES, LANES)
                    msg_v[row, sl] = msg_v[row, sl] * snorm
        pltpu.sync_copy(msg_v, agg_sh.at[dst2d.at[0]], add=True)
        return carry

    lax.fori_loop(0, NCHUNK, chunk, 0)
    plsc.subcore_barrier()
    nwriters = 10
    npiece = 25
    rows = N_NODES // nwriters // npiece

    @pl.when(s < nwriters)
    def _():
        for p in range(npiece):
            r0 = s * (N_NODES // nwriters) + p * rows
            pltpu.sync_copy(agg_sh.at[pl.ds(r0, rows)], out_bounce)
            pltpu.sync_copy(out_bounce,
                            agg_out.at[pl.ds(c * N_NODES + r0, rows)])


def _msg_call(xw, gidx, kidx, norm, zeros_nd):
    k = functools.partial(
        pl.kernel,
        out_type=jax.ShapeDtypeStruct((NUM_CORES * N_NODES, D_FEAT),
                                      jnp.float32),
        mesh=plsc.VectorSubcoreMesh(**_MESH),
        scratch_types=[
            pltpu.VMEM((CHUNK,), jnp.int32),
            pltpu.VMEM((CHUNK,), jnp.int32),
            pltpu.VMEM((CHUNK,), jnp.float32),
            pltpu.VMEM((1, CHUNK), jnp.int32),
            pltpu.VMEM((CHUNK, D_FEAT), jnp.float32),
            pltpu.VMEM((40, D_FEAT), jnp.float32),
            pltpu.VMEM_SHARED((NKEY,), jnp.float32),
            pltpu.VMEM_SHARED((N_NODES, D_FEAT), jnp.float32),
        ],
        compiler_params=_SC_PARAMS,
    )(_msg_body)
    return k(xw, gidx, kidx, norm, zeros_nd)


def _norm_body(cnt_ref, out_ref):
    cc = cnt_ref[...]
    out_ref[...] = 1.0 / jnp.maximum(cc[0] + cc[1], 1.0)


def _norm_call(cnt):
    cnt3 = cnt.reshape(NUM_CORES, NKEY // D_FEAT, D_FEAT)
    out = pl.pallas_call(
        _norm_body,
        out_shape=jax.ShapeDtypeStruct((NKEY // D_FEAT, D_FEAT), jnp.float32),
    )(cnt3)
    return out.reshape(NKEY)


def _dense_body(x_ref, comp_ref, basis_ref, root_ref, bias_ref,
                xw_ref, xroot_ref):
    r = pl.program_id(1)
    w = comp_ref[r, 0] * basis_ref[0]
    for b in range(1, NB_BASES):
        w = w + comp_ref[r, b] * basis_ref[b]
    xblk = x_ref[...]
    xw_ref[...] = jnp.dot(xblk, w, preferred_element_type=jnp.float32)

    @pl.when(r == 0)
    def _():
        xroot_ref[...] = (
            jnp.dot(xblk, root_ref[...], preferred_element_type=jnp.float32)
            + bias_ref[...]
        )


def _dense_call(x, comp_l, basis_l, root_l, bias_l):
    return pl.pallas_call(
        _dense_body,
        grid=(NBLK, R_REL),
        in_specs=[
            pl.BlockSpec((BLKN, D_FEAT), lambda nb, r: (nb, 0)),
            pl.BlockSpec(memory_space=pltpu.SMEM),
            pl.BlockSpec((NB_BASES, D_FEAT, D_FEAT), lambda nb, r: (0, 0, 0)),
            pl.BlockSpec((D_FEAT, D_FEAT), lambda nb, r: (0, 0)),
            pl.BlockSpec((1, D_FEAT), lambda nb, r: (0, 0)),
        ],
        out_specs=[
            pl.BlockSpec((BLKN, D_FEAT), lambda nb, r: (r * NBLK + nb, 0)),
            pl.BlockSpec((BLKN, D_FEAT), lambda nb, r: (nb, 0)),
        ],
        out_shape=[
            jax.ShapeDtypeStruct((R_REL * N_NODES, D_FEAT), jnp.float32),
            jax.ShapeDtypeStruct((N_NODES, D_FEAT), jnp.float32),
        ],
    )(x, comp_l, basis_l, root_l, bias_l)


def _fuse_body(p_ref, xroot_ref, x_ref, o_ref):
    pre = p_ref[0] + p_ref[1] + xroot_ref[...]
    o_ref[...] = jnp.maximum(pre, 0.0) + x_ref[...]


def _fuse_call(parts, xroot, x):
    return pl.pallas_call(
        _fuse_body,
        grid=(NBLK,),
        in_specs=[
            pl.BlockSpec((NUM_CORES, BLKN, D_FEAT), lambda nb: (0, nb, 0)),
            pl.BlockSpec((BLKN, D_FEAT), lambda nb: (nb, 0)),
            pl.BlockSpec((BLKN, D_FEAT), lambda nb: (nb, 0)),
        ],
        out_specs=pl.BlockSpec((BLKN, D_FEAT), lambda nb: (nb, 0)),
        out_shape=jax.ShapeDtypeStruct((N_NODES, D_FEAT), jnp.float32),
    )(parts, xroot, x)


def kernel(x_drug, x_protein, edge_index, edge_type, offset_drug,
           offset_protein, comp, basis, root, bias):
    x = jnp.concatenate([x_drug, x_protein], axis=0)
    src = edge_index[0]
    dst = edge_index[1]
    et = edge_type
    gidx = et * N_NODES + src
    kidx = dst * R_REL + et
    zeros_nk = jnp.zeros((NKEY,), jnp.float32)
    zeros_nd = jnp.zeros((N_NODES, D_FEAT), jnp.float32)

    cnt = _count_call(kidx, zeros_nk)
    norm = _norm_call(cnt)

    def layer(x_c, params):
        comp_l, basis_l, root_l, bias_l = params
        xw, xroot = _dense_call(x_c, comp_l, basis_l, root_l,
                                bias_l.reshape(1, D_FEAT))
        parts = _msg_call(xw, gidx, kidx, norm, zeros_nd)
        x_n = _fuse_call(parts.reshape(NUM_CORES, N_NODES, D_FEAT),
                         xroot, x_c)
        return x_n, None

    x, _ = lax.scan(layer, x, (comp, basis, root, bias))

    out_drug = lax.dynamic_slice_in_dim(x, offset_drug, x_drug.shape[0])
    out_protein = lax.dynamic_slice_in_dim(x, offset_protein,
                                           x_protein.shape[0])
    return (out_drug, out_protein)


# speedup vs baseline: 27.3055x; 1.8663x over previous
import functools

import jax
import jax.numpy as jnp
from jax import lax
from jax.experimental import pallas as pl
from jax.experimental.pallas import tpu as pltpu
from jax.experimental.pallas import tpu_sc as plsc

N_NODES = 10000
E_EDGES = 320000
D_FEAT = 128
R_REL = 8
NB_BASES = 8
NKEY = N_NODES * R_REL

NUM_CORES = 2
NUM_SUBCORES = 16
NUM_WORKERS = NUM_CORES * NUM_SUBCORES
EPT = E_EDGES // NUM_WORKERS
CHUNK = 80
NCHUNK = EPT // CHUNK
LANES = 16

BLKN = 2000
NBLK = N_NODES // BLKN

_MESH = dict(core_axis_name="c", subcore_axis_name="s",
             num_cores=NUM_CORES, num_subcores=NUM_SUBCORES)
_SC_PARAMS = pltpu.CompilerParams(needs_layout_passes=False)


def _cnt_body(kidx_hbm, zeros_hbm, cnt_out, kidx2d, ones_v, cnt_bounce,
              cnt_sh):
    c = lax.axis_index("c")
    s = lax.axis_index("s")
    wid = c * NUM_SUBCORES + s
    for g in range(CHUNK // LANES):
        ones_v[pl.ds(g * LANES, LANES)] = jnp.full((LANES,), 1.0, jnp.float32)

    @pl.when(s == 0)
    def _():
        pltpu.sync_copy(zeros_hbm, cnt_sh)

    plsc.subcore_barrier()
    base = wid * EPT

    def chunk(j, carry):
        off = base + j * CHUNK
        pltpu.sync_copy(kidx_hbm.at[pl.ds(off, CHUNK)], kidx2d.at[0])
        pltpu.sync_copy(ones_v, cnt_sh.at[kidx2d.at[0]], add=True)
        return carry

    lax.fori_loop(0, NCHUNK, chunk, 0)
    plsc.subcore_barrier()
    per_tile = NKEY // NUM_SUBCORES
    pltpu.sync_copy(cnt_sh.at[pl.ds(s * per_tile, per_tile)], cnt_bounce)
    pltpu.sync_copy(cnt_bounce,
                    cnt_out.at[pl.ds(c * NKEY + s * per_tile, per_tile)])


def _count_call(kidx, zeros_nk):
    k = functools.partial(
        pl.kernel,
        out_type=jax.ShapeDtypeStruct((NUM_CORES * NKEY,), jnp.float32),
        mesh=plsc.VectorSubcoreMesh(**_MESH),
        scratch_types=[
            pltpu.VMEM((1, CHUNK), jnp.int32),
            pltpu.VMEM((CHUNK,), jnp.float32),
            pltpu.VMEM((NKEY // NUM_SUBCORES,), jnp.float32),
            pltpu.VMEM_SHARED((NKEY,), jnp.float32),
        ],
        compiler_params=_SC_PARAMS,
    )(_cnt_body)
    return k(kidx, zeros_nk)


def _msg_body(xw_hbm, gidx_hbm, kidx_hbm, norm_hbm, zeros_hbm, agg_out,
              gidx0, gidx1, kidx0, kidx1, nrm0, nrm1, dst0, dst1,
              msg0, msg1, out_bounce,
              sem_ig0, sem_ig1, sem_ik0, sem_ik1,
              sem_g0, sem_g1, sem_n0, sem_n1,
              normtab_sh, agg_sh):
    gidx_b = (gidx0, gidx1)
    kidx_b = (kidx0, kidx1)
    nrm_b = (nrm0, nrm1)
    dst_b = (dst0, dst1)
    msg_b = (msg0, msg1)
    sem_ig = (sem_ig0, sem_ig1)
    sem_ik = (sem_ik0, sem_ik1)
    sem_g = (sem_g0, sem_g1)
    sem_n = (sem_n0, sem_n1)

    c = lax.axis_index("c")
    s = lax.axis_index("s")
    wid = c * NUM_SUBCORES + s

    @pl.when(s == 0)
    def _():
        pltpu.sync_copy(norm_hbm, normtab_sh)
        pltpu.sync_copy(zeros_hbm, agg_sh)

    plsc.subcore_barrier()
    base = wid * EPT

    def issue_idx(j, b):
        off = base + j * CHUNK
        pltpu.async_copy(gidx_hbm.at[pl.ds(off, CHUNK)], gidx_b[b], sem_ig[b])
        pltpu.async_copy(kidx_hbm.at[pl.ds(off, CHUNK)], kidx_b[b], sem_ik[b])

    def wait_idx(j, b):
        off = base + j * CHUNK
        pltpu.make_async_copy(gidx_hbm.at[pl.ds(off, CHUNK)], gidx_b[b],
                              sem_ig[b]).wait()
        pltpu.make_async_copy(kidx_hbm.at[pl.ds(off, CHUNK)], kidx_b[b],
                              sem_ik[b]).wait()

    def issue_gather(b):
        pltpu.async_copy(xw_hbm.at[gidx_b[b]], msg_b[b], sem_g[b])
        pltpu.async_copy(normtab_sh.at[kidx_b[b]], nrm_b[b], sem_n[b])

    def wait_gather(b):
        pltpu.make_async_copy(xw_hbm.at[gidx_b[b]], msg_b[b], sem_g[b]).wait()
        pltpu.make_async_copy(normtab_sh.at[kidx_b[b]], nrm_b[b],
                              sem_n[b]).wait()

    def build_dst(b):
        for g in range(CHUNK // LANES):
            kv = kidx_b[b][pl.ds(g * LANES, LANES)]
            dst_b[b][0, pl.ds(g * LANES, LANES)] = kv >> 3

    def scale_scatter(b):
        for g in range(CHUNK // LANES):
            nv = nrm_b[b][pl.ds(g * LANES, LANES)]
            for i in range(LANES):
                snorm = nv[i]
                row = g * LANES + i
                for t in range(D_FEAT // LANES):
                    sl = pl.ds(t * LANES, LANES)
                    msg_b[b][row, sl] = msg_b[b][row, sl] * snorm
        pltpu.sync_copy(msg_b[b], agg_sh.at[dst_b[b].at[0]], add=True)

    issue_idx(0, 0)
    wait_idx(0, 0)
    issue_gather(0)
    issue_idx(1, 1)

    def step(j, b):
        jn = j + 1
        bn = 1 - b
        wait_idx(jn, bn)
        issue_gather(bn)
        wait_gather(b)
        build_dst(b)

        @pl.when(jn + 1 < NCHUNK)
        def _():
            issue_idx(jn + 1, b)

        scale_scatter(b)

    def dstep(jj, carry):
        step(jj * 2, 0)
        step(jj * 2 + 1, 1)
        return carry

    lax.fori_loop(0, (NCHUNK - 1) // 2, dstep, 0)
    wait_gather(0)
    build_dst(0)
    scale_scatter(0)
    plsc.subcore_barrier()
    nwriters = 10
    npiece = 25
    rows = N_NODES // nwriters // npiece

    @pl.when(s < nwriters)
    def _():
        for p in range(npiece):
            r0 = s * (N_NODES // nwriters) + p * rows
            pltpu.sync_copy(agg_sh.at[pl.ds(r0, rows)], out_bounce)
            pltpu.sync_copy(out_bounce,
                            agg_out.at[pl.ds(c * N_NODES + r0, rows)])


def _msg_call(xw, gidx, kidx, norm, zeros_nd):
    k = functools.partial(
        pl.kernel,
        out_type=jax.ShapeDtypeStruct((NUM_CORES * N_NODES, D_FEAT),
                                      jnp.float32),
        mesh=plsc.VectorSubcoreMesh(**_MESH),
        scratch_types=[
            pltpu.VMEM((CHUNK,), jnp.int32),
            pltpu.VMEM((CHUNK,), jnp.int32),
            pltpu.VMEM((CHUNK,), jnp.int32),
            pltpu.VMEM((CHUNK,), jnp.int32),
            pltpu.VMEM((CHUNK,), jnp.float32),
            pltpu.VMEM((CHUNK,), jnp.float32),
            pltpu.VMEM((1, CHUNK), jnp.int32),
            pltpu.VMEM((1, CHUNK), jnp.int32),
            pltpu.VMEM((CHUNK, D_FEAT), jnp.float32),
            pltpu.VMEM((CHUNK, D_FEAT), jnp.float32),
            pltpu.VMEM((40, D_FEAT), jnp.float32),
            pltpu.SemaphoreType.DMA,
            pltpu.SemaphoreType.DMA,
            pltpu.SemaphoreType.DMA,
            pltpu.SemaphoreType.DMA,
            pltpu.SemaphoreType.DMA,
            pltpu.SemaphoreType.DMA,
            pltpu.SemaphoreType.DMA,
            pltpu.SemaphoreType.DMA,
            pltpu.VMEM_SHARED((NKEY,), jnp.float32),
            pltpu.VMEM_SHARED((N_NODES, D_FEAT), jnp.float32),
        ],
        compiler_params=_SC_PARAMS,
    )(_msg_body)
    return k(xw, gidx, kidx, norm, zeros_nd)


def _norm_body(cnt_ref, out_ref):
    cc = cnt_ref[...]
    out_ref[...] = 1.0 / jnp.maximum(cc[0] + cc[1], 1.0)


def _norm_call(cnt):
    cnt3 = cnt.reshape(NUM_CORES, NKEY // D_FEAT, D_FEAT)
    out = pl.pallas_call(
        _norm_body,
        out_shape=jax.ShapeDtypeStruct((NKEY // D_FEAT, D_FEAT), jnp.float32),
    )(cnt3)
    return out.reshape(NKEY)


def _dense_body(x_ref, comp_ref, basis_ref, root_ref, bias_ref,
                xw_ref, xroot_ref):
    r = pl.program_id(1)
    w = comp_ref[r, 0] * basis_ref[0]
    for b in range(1, NB_BASES):
        w = w + comp_ref[r, b] * basis_ref[b]
    xblk = x_ref[...]
    xw_ref[...] = jnp.dot(xblk, w, preferred_element_type=jnp.float32)

    @pl.when(r == 0)
    def _():
        xroot_ref[...] = (
            jnp.dot(xblk, root_ref[...], preferred_element_type=jnp.float32)
            + bias_ref[...]
        )


def _dense_call(x, comp_l, basis_l, root_l, bias_l):
    return pl.pallas_call(
        _dense_body,
        grid=(NBLK, R_REL),
        in_specs=[
            pl.BlockSpec((BLKN, D_FEAT), lambda nb, r: (nb, 0)),
            pl.BlockSpec(memory_space=pltpu.SMEM),
            pl.BlockSpec((NB_BASES, D_FEAT, D_FEAT), lambda nb, r: (0, 0, 0)),
            pl.BlockSpec((D_FEAT, D_FEAT), lambda nb, r: (0, 0)),
            pl.BlockSpec((1, D_FEAT), lambda nb, r: (0, 0)),
        ],
        out_specs=[
            pl.BlockSpec((BLKN, D_FEAT), lambda nb, r: (r * NBLK + nb, 0)),
            pl.BlockSpec((BLKN, D_FEAT), lambda nb, r: (nb, 0)),
        ],
        out_shape=[
            jax.ShapeDtypeStruct((R_REL * N_NODES, D_FEAT), jnp.float32),
            jax.ShapeDtypeStruct((N_NODES, D_FEAT), jnp.float32),
        ],
    )(x, comp_l, basis_l, root_l, bias_l)


def _fuse_body(p_ref, xroot_ref, x_ref, o_ref):
    pre = p_ref[0] + p_ref[1] + xroot_ref[...]
    o_ref[...] = jnp.maximum(pre, 0.0) + x_ref[...]


def _fuse_call(parts, xroot, x):
    return pl.pallas_call(
        _fuse_body,
        grid=(NBLK,),
        in_specs=[
            pl.BlockSpec((NUM_CORES, BLKN, D_FEAT), lambda nb: (0, nb, 0)),
            pl.BlockSpec((BLKN, D_FEAT), lambda nb: (nb, 0)),
            pl.BlockSpec((BLKN, D_FEAT), lambda nb: (nb, 0)),
        ],
        out_specs=pl.BlockSpec((BLKN, D_FEAT), lambda nb: (nb, 0)),
        out_shape=jax.ShapeDtypeStruct((N_NODES, D_FEAT), jnp.float32),
    )(parts, xroot, x)


def kernel(x_drug, x_protein, edge_index, edge_type, offset_drug,
           offset_protein, comp, basis, root, bias):
    x = jnp.concatenate([x_drug, x_protein], axis=0)
    src = edge_index[0]
    dst = edge_index[1]
    et = edge_type
    gidx = et * N_NODES + src
    kidx = dst * R_REL + et
    zeros_nk = jnp.zeros((NKEY,), jnp.float32)
    zeros_nd = jnp.zeros((N_NODES, D_FEAT), jnp.float32)

    cnt = _count_call(kidx, zeros_nk)
    norm = _norm_call(cnt)

    def layer(x_c, params):
        comp_l, basis_l, root_l, bias_l = params
        xw, xroot = _dense_call(x_c, comp_l, basis_l, root_l,
                                bias_l.reshape(1, D_FEAT))
        parts = _msg_call(xw, gidx, kidx, norm, zeros_nd)
        x_n = _fuse_call(parts.reshape(NUM_CORES, N_NODES, D_FEAT),
                         xroot, x_c)
        return x_n, None

    x, _ = lax.scan(layer, x, (comp, basis, root, bias))

    out_drug = lax.dynamic_slice_in_dim(x, offset_drug, x_drug.shape[0])
    out_protein = lax.dynamic_slice_in_dim(x, offset_protein,
                                           x_protein.shape[0])
    return (out_drug, out_protein)


# speedup vs baseline: 30.5395x; 1.1184x over previous
import functools

import jax
import jax.numpy as jnp
from jax import lax
from jax.experimental import pallas as pl
from jax.experimental.pallas import tpu as pltpu
from jax.experimental.pallas import tpu_sc as plsc

N_NODES = 10000
E_EDGES = 320000
D_FEAT = 128
R_REL = 8
NB_BASES = 8
NKEY = N_NODES * R_REL

NUM_CORES = 2
NUM_SUBCORES = 16
NUM_WORKERS = NUM_CORES * NUM_SUBCORES
EPT = E_EDGES // NUM_WORKERS
CHUNK = 80
NCHUNK = EPT // CHUNK
LANES = 16

BLKN = 2000
NBLK = N_NODES // BLKN

_MESH = dict(core_axis_name="c", subcore_axis_name="s",
             num_cores=NUM_CORES, num_subcores=NUM_SUBCORES)
_SC_PARAMS = pltpu.CompilerParams(needs_layout_passes=False)


def _cnt_body(kidx2_hbm, zeros_hbm, cnt_out, kidx2d, ones_v, cnt_bounce,
              sem_s, cnt_sh):
    c = lax.axis_index("c")
    s = lax.axis_index("s")
    wid = c * NUM_SUBCORES + s
    for g in range(CHUNK // LANES):
        ones_v[pl.ds(g * LANES, LANES)] = jnp.full((LANES,), 1.0, jnp.float32)

    @pl.when(s == 0)
    def _():
        pltpu.sync_copy(zeros_hbm, cnt_sh)

    pltpu.sync_copy(kidx2_hbm.at[wid], kidx2d)
    plsc.subcore_barrier()

    nfire = 5

    def blk(q, carry):
        for u in range(nfire):
            pltpu.async_copy(ones_v, cnt_sh.at[kidx2d.at[q * nfire + u]],
                             sem_s, add=True)
        for u in range(nfire):
            pltpu.make_async_copy(ones_v,
                                  cnt_sh.at[kidx2d.at[q * nfire + u]],
                                  sem_s).wait()
        return carry

    lax.fori_loop(0, NCHUNK // nfire, blk, 0)
    plsc.subcore_barrier()
    per_tile = NKEY // NUM_SUBCORES
    pltpu.sync_copy(cnt_sh.at[pl.ds(s * per_tile, per_tile)], cnt_bounce)
    pltpu.sync_copy(cnt_bounce,
                    cnt_out.at[pl.ds(c * NKEY + s * per_tile, per_tile)])


def _count_call(kidx, zeros_nk):
    k = functools.partial(
        pl.kernel,
        out_type=jax.ShapeDtypeStruct((NUM_CORES * NKEY,), jnp.float32),
        mesh=plsc.VectorSubcoreMesh(**_MESH),
        scratch_types=[
            pltpu.VMEM((NCHUNK, CHUNK), jnp.int32),
            pltpu.VMEM((CHUNK,), jnp.float32),
            pltpu.VMEM((NKEY // NUM_SUBCORES,), jnp.float32),
            pltpu.SemaphoreType.DMA,
            pltpu.VMEM_SHARED((NKEY,), jnp.float32),
        ],
        compiler_params=_SC_PARAMS,
    )(_cnt_body)
    return k(kidx.reshape(NUM_WORKERS, NCHUNK, CHUNK), zeros_nk)


def _msg_body(xw_hbm, gidx_hbm, kidx_hbm, norm_hbm, zeros_hbm, agg_out,
              gidx0, gidx1, kidx0, kidx1, nrm0, nrm1, dst0, dst1,
              msg0, msg1, out_bounce,
              sem_ig0, sem_ig1, sem_ik0, sem_ik1,
              sem_g0, sem_g1, sem_n0, sem_n1, sem_s0, sem_s1,
              normtab_sh, agg_sh):
    gidx_b = (gidx0, gidx1)
    kidx_b = (kidx0, kidx1)
    nrm_b = (nrm0, nrm1)
    dst_b = (dst0, dst1)
    msg_b = (msg0, msg1)
    sem_ig = (sem_ig0, sem_ig1)
    sem_ik = (sem_ik0, sem_ik1)
    sem_g = (sem_g0, sem_g1)
    sem_n = (sem_n0, sem_n1)
    sem_s = (sem_s0, sem_s1)

    c = lax.axis_index("c")
    s = lax.axis_index("s")
    wid = c * NUM_SUBCORES + s

    @pl.when(s == 0)
    def _():
        pltpu.sync_copy(norm_hbm, normtab_sh)
        pltpu.sync_copy(zeros_hbm, agg_sh)

    plsc.subcore_barrier()
    base = wid * EPT

    def issue_idx(j, b):
        off = base + j * CHUNK
        pltpu.async_copy(gidx_hbm.at[pl.ds(off, CHUNK)], gidx_b[b], sem_ig[b])
        pltpu.async_copy(kidx_hbm.at[pl.ds(off, CHUNK)], kidx_b[b], sem_ik[b])

    def wait_idx(j, b):
        off = base + j * CHUNK
        pltpu.make_async_copy(gidx_hbm.at[pl.ds(off, CHUNK)], gidx_b[b],
                              sem_ig[b]).wait()
        pltpu.make_async_copy(kidx_hbm.at[pl.ds(off, CHUNK)], kidx_b[b],
                              sem_ik[b]).wait()

    def issue_gather(b):
        pltpu.async_copy(xw_hbm.at[gidx_b[b]], msg_b[b], sem_g[b])
        pltpu.async_copy(normtab_sh.at[kidx_b[b]], nrm_b[b], sem_n[b])

    def wait_gather(b):
        pltpu.make_async_copy(xw_hbm.at[gidx_b[b]], msg_b[b], sem_g[b]).wait()
        pltpu.make_async_copy(normtab_sh.at[kidx_b[b]], nrm_b[b],
                              sem_n[b]).wait()

    def build_dst(b):
        for g in range(CHUNK // LANES):
            kv = kidx_b[b][pl.ds(g * LANES, LANES)]
            dst_b[b][0, pl.ds(g * LANES, LANES)] = kv >> 3

    def scale(b):
        for g in range(CHUNK // LANES):
            nv = nrm_b[b][pl.ds(g * LANES, LANES)]
            for i in range(LANES):
                snorm = nv[i]
                row = g * LANES + i
                for t in range(D_FEAT // LANES):
                    sl = pl.ds(t * LANES, LANES)
                    msg_b[b][row, sl] = msg_b[b][row, sl] * snorm

    def issue_scatter(b):
        pltpu.async_copy(msg_b[b], agg_sh.at[dst_b[b].at[0]], sem_s[b],
                         add=True)

    def wait_scatter(b):
        pltpu.make_async_copy(msg_b[b], agg_sh.at[dst_b[b].at[0]],
                              sem_s[b]).wait()

    issue_idx(0, 0)
    wait_idx(0, 0)
    issue_gather(0)
    issue_idx(1, 1)

    def step(j, b):
        jn = j + 1
        bn = 1 - b
        wait_idx(jn, bn)

        @pl.when(j >= 1)
        def _():
            wait_scatter(bn)

        issue_gather(bn)
        wait_gather(b)
        build_dst(b)

        @pl.when(jn + 1 < NCHUNK)
        def _():
            issue_idx(jn + 1, b)

        scale(b)
        issue_scatter(b)

    def dstep(jj, carry):
        step(jj * 2, 0)
        step(jj * 2 + 1, 1)
        return carry

    lax.fori_loop(0, (NCHUNK - 1) // 2, dstep, 0)
    wait_scatter(1)
    wait_gather(0)
    build_dst(0)
    scale(0)
    issue_scatter(0)
    wait_scatter(0)
    plsc.subcore_barrier()
    nwriters = 10
    npiece = 25
    rows = N_NODES // nwriters // npiece

    @pl.when(s < nwriters)
    def _():
        for p in range(npiece):
            r0 = s * (N_NODES // nwriters) + p * rows
            pltpu.sync_copy(agg_sh.at[pl.ds(r0, rows)], out_bounce)
            pltpu.sync_copy(out_bounce,
                            agg_out.at[pl.ds(c * N_NODES + r0, rows)])


def _msg_call(xw, gidx, kidx, norm, zeros_nd):
    k = functools.partial(
        pl.kernel,
        out_type=jax.ShapeDtypeStruct((NUM_CORES * N_NODES, D_FEAT),
                                      jnp.float32),
        mesh=plsc.VectorSubcoreMesh(**_MESH),
        scratch_types=[
            pltpu.VMEM((CHUNK,), jnp.int32),
            pltpu.VMEM((CHUNK,), jnp.int32),
            pltpu.VMEM((CHUNK,), jnp.int32),
            pltpu.VMEM((CHUNK,), jnp.int32),
            pltpu.VMEM((CHUNK,), jnp.float32),
            pltpu.VMEM((CHUNK,), jnp.float32),
            pltpu.VMEM((1, CHUNK), jnp.int32),
            pltpu.VMEM((1, CHUNK), jnp.int32),
            pltpu.VMEM((CHUNK, D_FEAT), jnp.float32),
            pltpu.VMEM((CHUNK, D_FEAT), jnp.float32),
            pltpu.VMEM((40, D_FEAT), jnp.float32),
            pltpu.SemaphoreType.DMA,
            pltpu.SemaphoreType.DMA,
            pltpu.SemaphoreType.DMA,
            pltpu.SemaphoreType.DMA,
            pltpu.SemaphoreType.DMA,
            pltpu.SemaphoreType.DMA,
            pltpu.SemaphoreType.DMA,
            pltpu.SemaphoreType.DMA,
            pltpu.SemaphoreType.DMA,
            pltpu.SemaphoreType.DMA,
            pltpu.VMEM_SHARED((NKEY,), jnp.float32),
            pltpu.VMEM_SHARED((N_NODES, D_FEAT), jnp.float32),
        ],
        compiler_params=_SC_PARAMS,
    )(_msg_body)
    return k(xw, gidx, kidx, norm, zeros_nd)


def _norm_body(cnt_ref, out_ref):
    cc = cnt_ref[...]
    out_ref[...] = 1.0 / jnp.maximum(cc[0] + cc[1], 1.0)


def _norm_call(cnt):
    cnt3 = cnt.reshape(NUM_CORES, NKEY // D_FEAT, D_FEAT)
    out = pl.pallas_call(
        _norm_body,
        out_shape=jax.ShapeDtypeStruct((NKEY // D_FEAT, D_FEAT), jnp.float32),
    )(cnt3)
    return out.reshape(NKEY)


def _dense_body(x_ref, comp_ref, basis_ref, root_ref, bias_ref,
                xw_ref, xroot_ref):
    r = pl.program_id(1)
    w = comp_ref[r, 0] * basis_ref[0]
    for b in range(1, NB_BASES):
        w = w + comp_ref[r, b] * basis_ref[b]
    xblk = x_ref[...]
    xw_ref[...] = jnp.dot(xblk, w, preferred_element_type=jnp.float32)

    @pl.when(r == 0)
    def _():
        xroot_ref[...] = (
            jnp.dot(xblk, root_ref[...], preferred_element_type=jnp.float32)
            + bias_ref[...]
        )


def _dense_call(x, comp_l, basis_l, root_l, bias_l):
    return pl.pallas_call(
        _dense_body,
        grid=(NBLK, R_REL),
        in_specs=[
            pl.BlockSpec((BLKN, D_FEAT), lambda nb, r: (nb, 0)),
            pl.BlockSpec(memory_space=pltpu.SMEM),
            pl.BlockSpec((NB_BASES, D_FEAT, D_FEAT), lambda nb, r: (0, 0, 0)),
            pl.BlockSpec((D_FEAT, D_FEAT), lambda nb, r: (0, 0)),
            pl.BlockSpec((1, D_FEAT), lambda nb, r: (0, 0)),
        ],
        out_specs=[
            pl.BlockSpec((BLKN, D_FEAT), lambda nb, r: (r * NBLK + nb, 0)),
            pl.BlockSpec((BLKN, D_FEAT), lambda nb, r: (nb, 0)),
        ],
        out_shape=[
            jax.ShapeDtypeStruct((R_REL * N_NODES, D_FEAT), jnp.float32),
            jax.ShapeDtypeStruct((N_NODES, D_FEAT), jnp.float32),
        ],
    )(x, comp_l, basis_l, root_l, bias_l)


def _fuse_body(p_ref, xroot_ref, x_ref, o_ref):
    pre = p_ref[0] + p_ref[1] + xroot_ref[...]
    o_ref[...] = jnp.maximum(pre, 0.0) + x_ref[...]


def _fuse_call(parts, xroot, x):
    return pl.pallas_call(
        _fuse_body,
        grid=(NBLK,),
        in_specs=[
            pl.BlockSpec((NUM_CORES, BLKN, D_FEAT), lambda nb: (0, nb, 0)),
            pl.BlockSpec((BLKN, D_FEAT), lambda nb: (nb, 0)),
            pl.BlockSpec((BLKN, D_FEAT), lambda nb: (nb, 0)),
        ],
        out_specs=pl.BlockSpec((BLKN, D_FEAT), lambda nb: (nb, 0)),
        out_shape=jax.ShapeDtypeStruct((N_NODES, D_FEAT), jnp.float32),
    )(parts, xroot, x)


def kernel(x_drug, x_protein, edge_index, edge_type, offset_drug,
           offset_protein, comp, basis, root, bias):
    x = jnp.concatenate([x_drug, x_protein], axis=0)
    src = edge_index[0]
    dst = edge_index[1]
    et = edge_type
    gidx = et * N_NODES + src
    kidx = dst * R_REL + et
    zeros_nk = jnp.zeros((NKEY,), jnp.float32)
    zeros_nd = jnp.zeros((N_NODES, D_FEAT), jnp.float32)

    cnt = _count_call(kidx, zeros_nk)
    norm = _norm_call(cnt)

    def layer(x_c, params):
        comp_l, basis_l, root_l, bias_l = params
        xw, xroot = _dense_call(x_c, comp_l, basis_l, root_l,
                                bias_l.reshape(1, D_FEAT))
        parts = _msg_call(xw, gidx, kidx, norm, zeros_nd)
        x_n = _fuse_call(parts.reshape(NUM_CORES, N_NODES, D_FEAT),
                         xroot, x_c)
        return x_n, None

    x, _ = lax.scan(layer, x, (comp, basis, root, bias))

    out_drug = lax.dynamic_slice_in_dim(x, offset_drug, x_drug.shape[0])
    out_protein = lax.dynamic_slice_in_dim(x, offset_protein,
                                           x_protein.shape[0])
    return (out_drug, out_protein)
